# baseline (speedup 1.0000x reference)
import functools

import jax
import jax.numpy as jnp
from jax import lax
from jax.experimental import pallas as pl
from jax.experimental.pallas import tpu as pltpu
from jax.experimental.pallas import tpu_sc as plsc

N = 10000
C = 128
K = 32
DEG = K + 1

NW = 32
N_PAD = 10240
CH = 4
ROWS = CH * K
NPT0 = 256
NPT1 = 384
PAIR = NPT0 + NPT1
NCH0 = NPT0 // CH
NCH1 = NPT1 // CH
NCH_MAX = max(NCH0, NCH1)
NBUF = 4
MM_NB = 2048


def _mm_body(x_ref, w_ref, o_ref):
    h = lax.dot_general(x_ref[...], w_ref[...], (((0,), (1,)), ((), ())),
                        preferred_element_type=jnp.float32)
    o_ref[...] = jnp.maximum(h, 0.0)


def _matmul_relu(x2d, W):
    grid = (N_PAD // MM_NB,)
    return pl.pallas_call(
        _mm_body,
        grid=grid,
        in_specs=[
            pl.BlockSpec((C, MM_NB), lambda i: (0, i)),
            pl.BlockSpec((C, C), lambda i: (0, 0)),
        ],
        out_specs=pl.BlockSpec((MM_NB, C), lambda i: (i, 0)),
        out_shape=jax.ShapeDtypeStruct((N_PAD, C), jnp.float32),
    )(x2d, W)


def _tr_body(a_ref, b_ref, o_ref):
    o_ref[...] = a_ref[...].T + b_ref[...]


def _transpose_bias(a, bias2d):
    grid = (N_PAD // MM_NB,)
    return pl.pallas_call(
        _tr_body,
        grid=grid,
        in_specs=[
            pl.BlockSpec((MM_NB, C), lambda i: (i, 0)),
            pl.BlockSpec((C, 1), lambda i: (0, 0)),
        ],
        out_specs=pl.BlockSpec((C, MM_NB), lambda i: (0, i)),
        out_shape=jax.ShapeDtypeStruct((C, N_PAD), jnp.float32),
    )(a, bias2d)


def _sc_body(h_hbm, idx_hbm, out_hbm, idx_v, gbuf, sbuf, obuf,
             gsems, osems):
    cid = lax.axis_index("c")
    sid = lax.axis_index("s")
    base = sid * PAIR + cid * NPT0
    cbase = sid * (PAIR // CH) + cid * (NPT0 // CH)
    nch = jnp.where(cid == 0, NCH0, NCH1)
    pltpu.sync_copy(idx_hbm.at[pl.ds(cbase, NCH_MAX)], idx_v)

    def fetch(chunk, buf):
        pltpu.make_async_copy(h_hbm.at[idx_v.at[chunk]], gbuf.at[buf],
                              gsems.at[buf]).start()
        pltpu.make_async_copy(h_hbm.at[pl.ds(base + chunk * CH, CH)],
                              sbuf.at[buf], gsems.at[buf]).start()

    def wait_fetch(buf):
        pltpu.make_async_copy(h_hbm.at[idx_v.at[0]], gbuf.at[buf],
                              gsems.at[buf]).wait()
        pltpu.make_async_copy(h_hbm.at[pl.ds(0, CH)], sbuf.at[buf],
                              gsems.at[buf]).wait()

    def put(chunk, buf):
        pltpu.make_async_copy(obuf.at[buf],
                              out_hbm.at[pl.ds(base + chunk * CH, CH)],
                              osems.at[buf]).start()

    def wait_put(buf):
        pltpu.make_async_copy(obuf.at[buf], out_hbm.at[pl.ds(0, CH)],
                              osems.at[buf]).wait()

    def compute(buf):
        for cn in range(CH):
            for gh in range(2):
                gs = [gh * 4 + g for g in range(4)]
                accs = [sbuf[buf, cn, pl.ds(g * 16, 16)] for g in gs]
                for k in range(K):
                    r = cn * K + k
                    for j, g in enumerate(gs):
                        accs[j] = accs[j] + gbuf[buf, r, pl.ds(g * 16, 16)]
                for j, g in enumerate(gs):
                    obuf[buf, cn, pl.ds(g * 16, 16)] = accs[j] * (1.0 / DEG)

    for b in range(NBUF - 1):
        fetch(b, b)

    def body(i, carry):
        for b in range(NBUF):
            chunk = i * NBUF + b
            wait_fetch(b)

            @pl.when(chunk >= NBUF)
            def _():
                wait_put(b)

            compute(b)
            put(chunk, b)
            nxt = chunk + (NBUF - 1)

            @pl.when(nxt < nch)
            def _():
                fetch(nxt, (b + NBUF - 1) % NBUF)

        return carry

    lax.fori_loop(0, nch // NBUF, body, 0)
    for b in range(NBUF):
        wait_put(b)


_sc_aggregate = functools.partial(
    pl.kernel,
    out_type=jax.ShapeDtypeStruct((N_PAD, C), jnp.float32),
    mesh=plsc.VectorSubcoreMesh(core_axis_name="c", subcore_axis_name="s"),
    scratch_types=[
        pltpu.VMEM((NCH_MAX, ROWS), jnp.int32),
        pltpu.VMEM((NBUF, ROWS, C), jnp.float32),
        pltpu.VMEM((NBUF, CH, C), jnp.float32),
        pltpu.VMEM((NBUF, CH, C), jnp.float32),
        pltpu.SemaphoreType.DMA((NBUF,)),
        pltpu.SemaphoreType.DMA((NBUF,)),
    ],
)(_sc_body)


def kernel(x, x_0, edge_index, W, bias):
    del x_0
    x2d = x[0, :, :, 0]
    x2d = jnp.pad(x2d, ((0, 0), (0, N_PAD - N)))
    h = _matmul_relu(x2d, W)

    idx = edge_index[0, 0]
    idx = jnp.pad(idx, ((0, N_PAD - N), (0, 0)))
    idx = idx.reshape(N_PAD // CH, ROWS)

    aggr = _sc_aggregate(h, idx)

    bias2d = bias.reshape(C, 1)
    out = _transpose_bias(aggr, bias2d)
    return out[:, :N].reshape(1, C, N, 1)


# speedup vs baseline: 1.1714x; 1.1714x over previous
import functools

import jax
import jax.numpy as jnp
from jax import lax
from jax.experimental import pallas as pl
from jax.experimental.pallas import tpu as pltpu
from jax.experimental.pallas import tpu_sc as plsc

N = 10000
C = 128
K = 32
DEG = K + 1

NW = 32
N_PAD = 10240
CH = 4
ROWS = CH * K
NPT0 = 384
NPT1 = 256
PAIR = NPT0 + NPT1
NCH0 = NPT0 // CH
NCH1 = NPT1 // CH
NCH_MAX = max(NCH0, NCH1)
NBUF = 4
MM_NB = 2048


def _mm_body(x_ref, w_ref, o_ref):
    h = lax.dot_general(x_ref[...], w_ref[...], (((0,), (1,)), ((), ())),
                        preferred_element_type=jnp.float32)
    o_ref[...] = jnp.maximum(h, 0.0)


def _matmul_relu(x2d, W):
    grid = (N_PAD // MM_NB,)
    return pl.pallas_call(
        _mm_body,
        grid=grid,
        in_specs=[
            pl.BlockSpec((C, MM_NB), lambda i: (0, i)),
            pl.BlockSpec((C, C), lambda i: (0, 0)),
        ],
        out_specs=pl.BlockSpec((MM_NB, C), lambda i: (i, 0)),
        out_shape=jax.ShapeDtypeStruct((N_PAD, C), jnp.float32),
    )(x2d, W)


def _tr_body(a_ref, b_ref, o_ref):
    o_ref[...] = a_ref[...].T + b_ref[...]


def _transpose_bias(a, bias2d):
    grid = (N_PAD // MM_NB,)
    return pl.pallas_call(
        _tr_body,
        grid=grid,
        in_specs=[
            pl.BlockSpec((MM_NB, C), lambda i: (i, 0)),
            pl.BlockSpec((C, 1), lambda i: (0, 0)),
        ],
        out_specs=pl.BlockSpec((C, MM_NB), lambda i: (0, i)),
        out_shape=jax.ShapeDtypeStruct((C, N_PAD), jnp.float32),
    )(a, bias2d)


def _sc_body(h_hbm, idx_hbm, out_hbm, idx_v, gbuf, sbuf, obuf,
             gsems, osems):
    cid = lax.axis_index("c")
    sid = lax.axis_index("s")
    base = sid * PAIR + cid * NPT0
    cbase = sid * (PAIR // CH) + cid * (NPT0 // CH)
    nch = jnp.where(cid == 0, NCH0, NCH1)
    pltpu.sync_copy(idx_hbm.at[pl.ds(cbase, NCH_MAX)], idx_v)

    def fetch(chunk, buf):
        pltpu.make_async_copy(h_hbm.at[idx_v.at[chunk]], gbuf.at[buf],
                              gsems.at[buf]).start()
        pltpu.make_async_copy(h_hbm.at[pl.ds(base + chunk * CH, CH)],
                              sbuf.at[buf], gsems.at[buf]).start()

    def wait_fetch(buf):
        pltpu.make_async_copy(h_hbm.at[idx_v.at[0]], gbuf.at[buf],
                              gsems.at[buf]).wait()
        pltpu.make_async_copy(h_hbm.at[pl.ds(0, CH)], sbuf.at[buf],
                              gsems.at[buf]).wait()

    def put(chunk, buf):
        pltpu.make_async_copy(obuf.at[buf],
                              out_hbm.at[pl.ds(base + chunk * CH, CH)],
                              osems.at[buf]).start()

    def wait_put(buf):
        pltpu.make_async_copy(obuf.at[buf], out_hbm.at[pl.ds(0, CH)],
                              osems.at[buf]).wait()

    def compute(buf):
        for cn in range(CH):
            for gh in range(2):
                gs = [gh * 4 + g for g in range(4)]
                accs = [sbuf[buf, cn, pl.ds(g * 16, 16)] for g in gs]
                for k in range(K):
                    r = cn * K + k
                    for j, g in enumerate(gs):
                        accs[j] = accs[j] + gbuf[buf, r, pl.ds(g * 16, 16)]
                for j, g in enumerate(gs):
                    obuf[buf, cn, pl.ds(g * 16, 16)] = accs[j] * (1.0 / DEG)

    for b in range(NBUF - 1):
        fetch(b, b)

    def body(i, carry):
        for b in range(NBUF):
            chunk = i * NBUF + b
            wait_fetch(b)

            @pl.when(chunk >= NBUF)
            def _():
                wait_put(b)

            compute(b)
            put(chunk, b)
            nxt = chunk + (NBUF - 1)

            @pl.when(nxt < nch)
            def _():
                fetch(nxt, (b + NBUF - 1) % NBUF)

        return carry

    lax.fori_loop(0, nch // NBUF, body, 0)
    for b in range(NBUF):
        wait_put(b)


_sc_aggregate = functools.partial(
    pl.kernel,
    out_type=jax.ShapeDtypeStruct((N_PAD, C), jnp.float32),
    mesh=plsc.VectorSubcoreMesh(core_axis_name="c", subcore_axis_name="s"),
    scratch_types=[
        pltpu.VMEM((NCH_MAX, ROWS), jnp.int32),
        pltpu.VMEM((NBUF, ROWS, C), jnp.float32),
        pltpu.VMEM((NBUF, CH, C), jnp.float32),
        pltpu.VMEM((NBUF, CH, C), jnp.float32),
        pltpu.SemaphoreType.DMA((NBUF,)),
        pltpu.SemaphoreType.DMA((NBUF,)),
    ],
)(_sc_body)


def kernel(x, x_0, edge_index, W, bias):
    del x_0
    x2d = x[0, :, :, 0]
    x2d = jnp.pad(x2d, ((0, 0), (0, N_PAD - N)))
    h = _matmul_relu(x2d, W)

    idx = edge_index[0, 0]
    idx = jnp.pad(idx, ((0, N_PAD - N), (0, 0)))
    idx = idx.reshape(N_PAD // CH, ROWS)

    aggr = _sc_aggregate(h, idx)

    bias2d = bias.reshape(C, 1)
    out = _transpose_bias(aggr, bias2d)
    return out[:, :N].reshape(1, C, N, 1)


# speedup vs baseline: 1.1980x; 1.0228x over previous
import functools

import jax
import jax.numpy as jnp
from jax import lax
from jax.experimental import pallas as pl
from jax.experimental.pallas import tpu as pltpu
from jax.experimental.pallas import tpu_sc as plsc

N = 10000
C = 128
K = 32
DEG = K + 1

NW = 32
N_PAD = 10240
CH = 4
ROWS = CH * K
NPT0 = 416
NPT1 = 224
PAIR = NPT0 + NPT1
NCH0 = NPT0 // CH
NCH1 = NPT1 // CH
NCH_MAX = max(NCH0, NCH1)
NBUF = 4
MM_NB = 2048


def _mm_body(x_ref, w_ref, o_ref):
    h = lax.dot_general(x_ref[...], w_ref[...], (((0,), (1,)), ((), ())),
                        preferred_element_type=jnp.float32)
    o_ref[...] = jnp.maximum(h, 0.0)


def _matmul_relu(x2d, W):
    grid = (N_PAD // MM_NB,)
    return pl.pallas_call(
        _mm_body,
        grid=grid,
        in_specs=[
            pl.BlockSpec((C, MM_NB), lambda i: (0, i)),
            pl.BlockSpec((C, C), lambda i: (0, 0)),
        ],
        out_specs=pl.BlockSpec((MM_NB, C), lambda i: (i, 0)),
        out_shape=jax.ShapeDtypeStruct((N_PAD, C), jnp.float32),
    )(x2d, W)


def _tr_body(a_ref, b_ref, o_ref):
    o_ref[...] = a_ref[...].T + b_ref[...]


def _transpose_bias(a, bias2d):
    grid = (N_PAD // MM_NB,)
    return pl.pallas_call(
        _tr_body,
        grid=grid,
        in_specs=[
            pl.BlockSpec((MM_NB, C), lambda i: (i, 0)),
            pl.BlockSpec((C, 1), lambda i: (0, 0)),
        ],
        out_specs=pl.BlockSpec((C, MM_NB), lambda i: (0, i)),
        out_shape=jax.ShapeDtypeStruct((C, N_PAD), jnp.float32),
    )(a, bias2d)


def _sc_body(h_hbm, idx_hbm, out_hbm, idx_v, gbuf, sbuf, obuf,
             gsems, osems):
    cid = lax.axis_index("c")
    sid = lax.axis_index("s")
    base = sid * PAIR + cid * NPT0
    cbase = sid * (PAIR // CH) + cid * (NPT0 // CH)
    nch = jnp.where(cid == 0, NCH0, NCH1)
    pltpu.sync_copy(idx_hbm.at[pl.ds(cbase, NCH_MAX)], idx_v)

    def fetch(chunk, buf):
        pltpu.make_async_copy(h_hbm.at[idx_v.at[chunk]], gbuf.at[buf],
                              gsems.at[buf]).start()
        pltpu.make_async_copy(h_hbm.at[pl.ds(base + chunk * CH, CH)],
                              sbuf.at[buf], gsems.at[buf]).start()

    def wait_fetch(buf):
        pltpu.make_async_copy(h_hbm.at[idx_v.at[0]], gbuf.at[buf],
                              gsems.at[buf]).wait()
        pltpu.make_async_copy(h_hbm.at[pl.ds(0, CH)], sbuf.at[buf],
                              gsems.at[buf]).wait()

    def put(chunk, buf):
        pltpu.make_async_copy(obuf.at[buf],
                              out_hbm.at[pl.ds(base + chunk * CH, CH)],
                              osems.at[buf]).start()

    def wait_put(buf):
        pltpu.make_async_copy(obuf.at[buf], out_hbm.at[pl.ds(0, CH)],
                              osems.at[buf]).wait()

    def compute(buf):
        for cn in range(CH):
            for gh in range(2):
                gs = [gh * 4 + g for g in range(4)]
                accs = [sbuf[buf, cn, pl.ds(g * 16, 16)] for g in gs]
                for k in range(K):
                    r = cn * K + k
                    for j, g in enumerate(gs):
                        accs[j] = accs[j] + gbuf[buf, r, pl.ds(g * 16, 16)]
                for j, g in enumerate(gs):
                    obuf[buf, cn, pl.ds(g * 16, 16)] = accs[j] * (1.0 / DEG)

    for b in range(NBUF - 1):
        fetch(b, b)

    def body(i, carry):
        for b in range(NBUF):
            chunk = i * NBUF + b
            wait_fetch(b)

            @pl.when(chunk >= NBUF)
            def _():
                wait_put(b)

            compute(b)
            put(chunk, b)
            nxt = chunk + (NBUF - 1)

            @pl.when(nxt < nch)
            def _():
                fetch(nxt, (b + NBUF - 1) % NBUF)

        return carry

    lax.fori_loop(0, nch // NBUF, body, 0)
    for b in range(NBUF):
        wait_put(b)


_sc_aggregate = functools.partial(
    pl.kernel,
    out_type=jax.ShapeDtypeStruct((N_PAD, C), jnp.float32),
    mesh=plsc.VectorSubcoreMesh(core_axis_name="c", subcore_axis_name="s"),
    scratch_types=[
        pltpu.VMEM((NCH_MAX, ROWS), jnp.int32),
        pltpu.VMEM((NBUF, ROWS, C), jnp.float32),
        pltpu.VMEM((NBUF, CH, C), jnp.float32),
        pltpu.VMEM((NBUF, CH, C), jnp.float32),
        pltpu.SemaphoreType.DMA((NBUF,)),
        pltpu.SemaphoreType.DMA((NBUF,)),
    ],
)(_sc_body)


def kernel(x, x_0, edge_index, W, bias):
    del x_0
    x2d = x[0, :, :, 0]
    x2d = jnp.pad(x2d, ((0, 0), (0, N_PAD - N)))
    h = _matmul_relu(x2d, W)

    idx = edge_index[0, 0]
    idx = jnp.pad(idx, ((0, N_PAD - N), (0, 0)))
    idx = idx.reshape(N_PAD // CH, ROWS)

    aggr = _sc_aggregate(h, idx)

    bias2d = bias.reshape(C, 1)
    out = _transpose_bias(aggr, bias2d)
    return out[:, :N].reshape(1, C, N, 1)
